# baseline (speedup 1.0000x reference)
import functools

import jax
import jax.numpy as jnp
from jax import lax
from jax.experimental import pallas as pl
from jax.experimental.pallas import tpu as pltpu
from jax.experimental.pallas import tpu_sc as plsc

_NUM_BINS = 64
_MAX_DIST = 25.0
_HIDDEN = 1024
_N = 4096
_NB = 16
_NC = 2
_NS = 16
_NW = _NC * _NS
_L = 16
_NP = _N + _L
_HB = _NB * _NUM_BINS


_MAX2 = _MAX_DIST * _MAX_DIST


def _sc_hist(xs, ys, zs, batch, starts, edges2):
    mesh = plsc.VectorSubcoreMesh(core_axis_name="c", subcore_axis_name="s")

    @functools.partial(
        pl.kernel,
        mesh=mesh,
        out_type=jax.ShapeDtypeStruct((_NW, _HB), jnp.float32),
        compiler_params=pltpu.CompilerParams(needs_layout_passes=False),
        scratch_types=[
            pltpu.VMEM((_NP,), jnp.float32),
            pltpu.VMEM((_NP,), jnp.float32),
            pltpu.VMEM((_NP,), jnp.float32),
            pltpu.VMEM((_NP,), jnp.float32),
            pltpu.VMEM((_NP,), jnp.int32),
            pltpu.VMEM((32,), jnp.int32),
            pltpu.VMEM((_NUM_BINS,), jnp.float32),
            pltpu.VMEM((_L * _HB,), jnp.float32),
            pltpu.VMEM((_HB,), jnp.float32),
        ],
    )
    def hist_kernel(xs_h, ys_h, zs_h, batch_h, starts_h, edges_h, out_h,
                    xv, yv, zv, sqv, bv, sv, ev, hist, red):
        wid = lax.axis_index("s") * _NC + lax.axis_index("c")
        pltpu.sync_copy(xs_h, xv)
        pltpu.sync_copy(ys_h, yv)
        pltpu.sync_copy(zs_h, zv)
        pltpu.sync_copy(batch_h, bv)
        pltpu.sync_copy(starts_h, sv)
        pltpu.sync_copy(edges_h, ev)

        zeros = jnp.zeros((_L,), jnp.float32)
        lanes = lax.iota(jnp.int32, _L)

        def sq_body(c, _):
            x = xv[pl.ds(c * _L, _L)]
            y = yv[pl.ds(c * _L, _L)]
            z = zv[pl.ds(c * _L, _L)]
            sqv[pl.ds(c * _L, _L)] = x * x + y * y + z * z
            return _
        lax.fori_loop(0, _NP // _L, sq_body, None)

        def zero_body(c, _):
            hist[pl.ds(c * _L, _L)] = zeros
            return _
        lax.fori_loop(0, (_L * _HB) // _L, zero_body, None)

        lanebase = lanes * _HB
        ones = jnp.ones((_L,), jnp.float32)

        def row_body(r, _):
            i = r * _NW + wid
            xi = xv[pl.ds(i, _L)][0]
            yi = yv[pl.ds(i, _L)][0]
            zi = zv[pl.ds(i, _L)][0]
            sqi = sqv[pl.ds(i, _L)][0]
            g = bv[pl.ds(i, _L)][0]
            jend = sv[pl.ds(g + 1, _L)][0]
            rowbase = lanebase + g * _NUM_BINS
            c0 = (i + 1) // _L
            c1 = (jend + (_L - 1)) // _L
            nk = (c1 - c0 + 1) // 2

            def do_chunk(c):
                j0 = c * _L
                jv_ = j0 + lanes
                xj = xv[pl.ds(j0, _L)]
                yj = yv[pl.ds(j0, _L)]
                zj = zv[pl.ds(j0, _L)]
                sqj = sqv[pl.ds(j0, _L)]
                dot = xi * xj + yi * yj + zi * zj
                d2 = jnp.maximum(sqi + sqj - 2.0 * dot, 0.0)
                b = jnp.zeros((_L,), jnp.int32)
                for step in (32, 16, 8, 4, 2, 1):
                    cand = b + step
                    e = plsc.load_gather(ev, [cand])
                    b = jnp.where(d2 >= e, cand, b)
                mask = (jv_ > i) & (jv_ < jend) & (d2 <= _MAX2)
                plsc.addupdate_scatter(hist, [rowbase + b], ones, mask=mask)

            def chunk_body(k, _):
                c = c0 + 2 * k
                do_chunk(c)
                do_chunk(c + 1)
                return _

            lax.fori_loop(0, nk, chunk_body, None)
            return _

        lax.fori_loop(0, _N // _NW, row_body, None)

        def red_body(c, _):
            acc = hist[pl.ds(c * _L, _L)]
            for l in range(1, _L):
                acc = acc + hist[pl.ds(l * _HB + c * _L, _L)]
            red[pl.ds(c * _L, _L)] = acc
            return _
        lax.fori_loop(0, _HB // _L, red_body, None)
        pltpu.sync_copy(red, out_h.at[wid])

    return hist_kernel(xs, ys, zs, batch, starts, edges2)


def _mlp_body(p_ref, w1_ref, b1_ref, w2_ref, b2_ref, o_ref):
    p = p_ref[...]
    cols = lax.broadcasted_iota(jnp.int32, (_NB, _NW * _NB), 1)
    rows = lax.broadcasted_iota(jnp.int32, (_NB, _NW * _NB), 0)
    sel = jnp.where((cols % _NB) == rows, 1.0, 0.0)
    hist = lax.dot_general(sel, p, (((1,), (0,)), ((), ())),
                           precision=lax.Precision.HIGHEST,
                           preferred_element_type=jnp.float32)
    hist = hist / (jnp.sum(hist, axis=1, keepdims=True) + 1e-8)
    h = lax.dot_general(hist, w1_ref[...], (((1,), (1,)), ((), ())),
                        precision=lax.Precision.HIGHEST,
                        preferred_element_type=jnp.float32) + b1_ref[...]
    h = h * (1.0 / (1.0 + jnp.exp(-h)))
    o = lax.dot_general(h, w2_ref[...], (((1,), (1,)), ((), ())),
                        precision=lax.Precision.HIGHEST,
                        preferred_element_type=jnp.float32) + b2_ref[...]
    o_ref[...] = o


def kernel(pos, batch, W1, b1, W2, b2):
    pad = jnp.zeros((_L,), jnp.float32)
    xs = jnp.concatenate([pos[:, 0], pad])
    ys = jnp.concatenate([pos[:, 1], pad])
    zs = jnp.concatenate([pos[:, 2], pad])
    batch_p = jnp.concatenate([batch, jnp.zeros((_L,), jnp.int32)])
    starts = jnp.searchsorted(batch, jnp.arange(_NB + 1, dtype=jnp.int32))
    starts = jnp.concatenate(
        [starts.astype(jnp.int32), jnp.zeros((32 - (_NB + 1),), jnp.int32)])
    edges2 = jnp.array([(b * _MAX_DIST / _NUM_BINS) ** 2
                        for b in range(_NUM_BINS)], jnp.float32)
    parts = _sc_hist(xs, ys, zs, batch_p, starts, edges2)
    parts2 = parts.reshape(_NW * _NB, _NUM_BINS)
    out = pl.pallas_call(
        _mlp_body,
        out_shape=jax.ShapeDtypeStruct((_NB, _HIDDEN), jnp.float32),
    )(parts2, W1, b1.reshape(1, _HIDDEN), W2, b2.reshape(1, _HIDDEN))
    return out


# speedup vs baseline: 1.4735x; 1.4735x over previous
import functools

import jax
import jax.numpy as jnp
from jax import lax
from jax.experimental import pallas as pl
from jax.experimental.pallas import tpu as pltpu
from jax.experimental.pallas import tpu_sc as plsc

_NUM_BINS = 64
_MAX_DIST = 25.0
_HIDDEN = 1024
_N = 4096
_NB = 16
_NC = 2
_NS = 16
_NW = _NC * _NS
_L = 16
_NP = _N + _L
_HB = _NB * _NUM_BINS


_MAX2 = _MAX_DIST * _MAX_DIST


def _sc_hist(xs, ys, zs, batch, starts, edges2):
    mesh = plsc.VectorSubcoreMesh(core_axis_name="c", subcore_axis_name="s")

    @functools.partial(
        pl.kernel,
        mesh=mesh,
        out_type=jax.ShapeDtypeStruct((_NW, _HB), jnp.float32),
        compiler_params=pltpu.CompilerParams(needs_layout_passes=False),
        scratch_types=[
            pltpu.VMEM((_NP,), jnp.float32),
            pltpu.VMEM((_NP,), jnp.float32),
            pltpu.VMEM((_NP,), jnp.float32),
            pltpu.VMEM((_NP,), jnp.float32),
            pltpu.VMEM((_NP,), jnp.int32),
            pltpu.VMEM((32,), jnp.int32),
            pltpu.VMEM((_NUM_BINS,), jnp.float32),
            pltpu.VMEM((_L * _HB,), jnp.float32),
            pltpu.VMEM((_HB,), jnp.float32),
        ],
    )
    def hist_kernel(xs_h, ys_h, zs_h, batch_h, starts_h, edges_h, out_h,
                    xv, yv, zv, sqv, bv, sv, ev, hist, red):
        wid = lax.axis_index("s") * _NC + lax.axis_index("c")
        pltpu.sync_copy(xs_h, xv)
        pltpu.sync_copy(ys_h, yv)
        pltpu.sync_copy(zs_h, zv)
        pltpu.sync_copy(batch_h, bv)
        pltpu.sync_copy(starts_h, sv)
        pltpu.sync_copy(edges_h, ev)

        zeros = jnp.zeros((_L,), jnp.float32)
        lanes = lax.iota(jnp.int32, _L)

        def sq_body(c, _):
            x = xv[pl.ds(c * _L, _L)]
            y = yv[pl.ds(c * _L, _L)]
            z = zv[pl.ds(c * _L, _L)]
            sqv[pl.ds(c * _L, _L)] = x * x + y * y + z * z
            return _
        lax.fori_loop(0, _NP // _L, sq_body, None)

        def zero_body(c, _):
            hist[pl.ds(c * _L, _L)] = zeros
            return _
        lax.fori_loop(0, (_L * _HB) // _L, zero_body, None)

        lanebase = lanes * _HB
        ones = jnp.ones((_L,), jnp.float32)

        def row_body(r, _):
            i = r * _NW + wid
            xi = xv[pl.ds(i, _L)][0]
            yi = yv[pl.ds(i, _L)][0]
            zi = zv[pl.ds(i, _L)][0]
            sqi = sqv[pl.ds(i, _L)][0]
            g = bv[pl.ds(i, _L)][0]
            jend = sv[pl.ds(g + 1, _L)][0]
            rowbase = lanebase + g * _NUM_BINS
            c0 = (i + 1) // _L
            c1 = (jend + (_L - 1)) // _L

            @plsc.parallel_loop(c0, c1, 1, unroll=4)
            def do_chunk(c):
                j0 = c * _L
                jv_ = j0 + lanes
                xj = xv[pl.ds(j0, _L)]
                yj = yv[pl.ds(j0, _L)]
                zj = zv[pl.ds(j0, _L)]
                sqj = sqv[pl.ds(j0, _L)]
                dot = xi * xj + yi * yj + zi * zj
                d2 = jnp.maximum(sqi + sqj - 2.0 * dot, 0.0)
                b = jnp.zeros((_L,), jnp.int32)
                for step in (32, 16, 8, 4, 2, 1):
                    cand = b + step
                    e = plsc.load_gather(ev, [cand])
                    b = jnp.where(d2 >= e, cand, b)
                mask = (jv_ > i) & (jv_ < jend) & (d2 <= _MAX2)
                plsc.addupdate_scatter(hist, [rowbase + b], ones, mask=mask)

            return _

        lax.fori_loop(0, _N // _NW, row_body, None)

        def red_body(c, _):
            acc = hist[pl.ds(c * _L, _L)]
            for l in range(1, _L):
                acc = acc + hist[pl.ds(l * _HB + c * _L, _L)]
            red[pl.ds(c * _L, _L)] = acc
            return _
        lax.fori_loop(0, _HB // _L, red_body, None)
        pltpu.sync_copy(red, out_h.at[wid])

    return hist_kernel(xs, ys, zs, batch, starts, edges2)


def _mlp_body(p_ref, w1_ref, b1_ref, w2_ref, b2_ref, o_ref):
    p = p_ref[...]
    cols = lax.broadcasted_iota(jnp.int32, (_NB, _NW * _NB), 1)
    rows = lax.broadcasted_iota(jnp.int32, (_NB, _NW * _NB), 0)
    sel = jnp.where((cols % _NB) == rows, 1.0, 0.0)
    hist = lax.dot_general(sel, p, (((1,), (0,)), ((), ())),
                           precision=lax.Precision.HIGHEST,
                           preferred_element_type=jnp.float32)
    hist = hist / (jnp.sum(hist, axis=1, keepdims=True) + 1e-8)
    h = lax.dot_general(hist, w1_ref[...], (((1,), (1,)), ((), ())),
                        precision=lax.Precision.HIGHEST,
                        preferred_element_type=jnp.float32) + b1_ref[...]
    h = h * (1.0 / (1.0 + jnp.exp(-h)))
    o = lax.dot_general(h, w2_ref[...], (((1,), (1,)), ((), ())),
                        precision=lax.Precision.HIGHEST,
                        preferred_element_type=jnp.float32) + b2_ref[...]
    o_ref[...] = o


def kernel(pos, batch, W1, b1, W2, b2):
    pad = jnp.zeros((_L,), jnp.float32)
    xs = jnp.concatenate([pos[:, 0], pad])
    ys = jnp.concatenate([pos[:, 1], pad])
    zs = jnp.concatenate([pos[:, 2], pad])
    batch_p = jnp.concatenate([batch, jnp.zeros((_L,), jnp.int32)])
    starts = jnp.searchsorted(batch, jnp.arange(_NB + 1, dtype=jnp.int32))
    starts = jnp.concatenate(
        [starts.astype(jnp.int32), jnp.zeros((32 - (_NB + 1),), jnp.int32)])
    edges2 = jnp.array([(b * _MAX_DIST / _NUM_BINS) ** 2
                        for b in range(_NUM_BINS)], jnp.float32)
    parts = _sc_hist(xs, ys, zs, batch_p, starts, edges2)
    parts2 = parts.reshape(_NW * _NB, _NUM_BINS)
    out = pl.pallas_call(
        _mlp_body,
        out_shape=jax.ShapeDtypeStruct((_NB, _HIDDEN), jnp.float32),
    )(parts2, W1, b1.reshape(1, _HIDDEN), W2, b2.reshape(1, _HIDDEN))
    return out


# speedup vs baseline: 1.7991x; 1.2210x over previous
import functools

import jax
import jax.numpy as jnp
from jax import lax
from jax.experimental import pallas as pl
from jax.experimental.pallas import tpu as pltpu
from jax.experimental.pallas import tpu_sc as plsc

_NUM_BINS = 64
_MAX_DIST = 25.0
_HIDDEN = 1024
_N = 4096
_NB = 16
_NC = 2
_NS = 16
_NW = _NC * _NS
_L = 16
_NP = _N + _L
_HB = _NB * _NUM_BINS
_MAX2 = _MAX_DIST * _MAX_DIST


def _sc_hist(pos_flat, batch, starts, edges2):
    mesh = plsc.VectorSubcoreMesh(core_axis_name="c", subcore_axis_name="s")

    @functools.partial(
        pl.kernel,
        mesh=mesh,
        out_type=jax.ShapeDtypeStruct((_NW, _HB), jnp.float32),
        compiler_params=pltpu.CompilerParams(needs_layout_passes=False),
        scratch_types=[
            pltpu.VMEM((3 * _NP,), jnp.float32),
            pltpu.VMEM((_NP,), jnp.int32),
            pltpu.VMEM((32,), jnp.int32),
            pltpu.VMEM((_NUM_BINS,), jnp.float32),
            pltpu.VMEM((_L * _HB,), jnp.float32),
            pltpu.VMEM((_HB,), jnp.float32),
        ],
    )
    def hist_kernel(pos_h, batch_h, starts_h, edges_h, out_h,
                    pv, bv, sv, ev, hist, red):
        wid = lax.axis_index("s") * _NC + lax.axis_index("c")
        pltpu.sync_copy(pos_h, pv)
        pltpu.sync_copy(batch_h, bv)
        pltpu.sync_copy(starts_h, sv)
        pltpu.sync_copy(edges_h, ev)

        zeros = jnp.zeros((_L,), jnp.float32)
        lanes = lax.iota(jnp.int32, _L)

        @plsc.parallel_loop(0, (_L * _HB) // _L, 1, unroll=8)
        def zero_body(c):
            hist[pl.ds(c * _L, _L)] = zeros

        lanebase = lanes * _HB
        ones = jnp.ones((_L,), jnp.float32)

        def row_body(r, _):
            i = r * _NW + wid
            pvec = pv[pl.ds(3 * i, _L)]
            xi = pvec[0]
            yi = pvec[1]
            zi = pvec[2]
            sqi = xi * xi + yi * yi + zi * zi
            g = bv[pl.ds(i, _L)][0]
            jend = sv[pl.ds(g + 1, _L)][0]
            rowbase = lanebase + g * _NUM_BINS
            c0 = (i + 1) // _L
            c1 = (jend + (_L - 1)) // _L

            @plsc.parallel_loop(c0, c1, 1, unroll=4)
            def do_chunk(c):
                j0 = c * _L
                jv_ = j0 + lanes
                j3 = jv_ * 3
                xj = plsc.load_gather(pv, [j3])
                yj = plsc.load_gather(pv, [j3 + 1])
                zj = plsc.load_gather(pv, [j3 + 2])
                sqj = xj * xj + yj * yj + zj * zj
                dot = xi * xj + yi * yj + zi * zj
                d2 = jnp.maximum(sqi + sqj - 2.0 * dot, 0.0)
                b = jnp.zeros((_L,), jnp.int32)
                for step in (32, 16, 8, 4, 2, 1):
                    cand = b + step
                    e = plsc.load_gather(ev, [cand])
                    b = jnp.where(d2 >= e, cand, b)
                mask = (jv_ > i) & (jv_ < jend) & (d2 <= _MAX2)
                plsc.addupdate_scatter(hist, [rowbase + b], ones, mask=mask)

            return _

        lax.fori_loop(0, _N // _NW, row_body, None)

        @plsc.parallel_loop(0, _HB // _L, 1, unroll=2)
        def red_body(c):
            acc = hist[pl.ds(c * _L, _L)]
            for l in range(1, _L):
                acc = acc + hist[pl.ds(l * _HB + c * _L, _L)]
            red[pl.ds(c * _L, _L)] = acc

        pltpu.sync_copy(red, out_h.at[wid])

    return hist_kernel(pos_flat, batch, starts, edges2)


def _mlp_body(p_ref, w1_ref, b1_ref, w2_ref, b2_ref, o_ref):
    p = p_ref[...]
    cols = lax.broadcasted_iota(jnp.int32, (_NB, _NW * _NB), 1)
    rows = lax.broadcasted_iota(jnp.int32, (_NB, _NW * _NB), 0)
    sel = jnp.where((cols % _NB) == rows, 1.0, 0.0)
    hist = lax.dot_general(sel, p, (((1,), (0,)), ((), ())),
                           precision=lax.Precision.HIGHEST,
                           preferred_element_type=jnp.float32)
    hist = hist / (jnp.sum(hist, axis=1, keepdims=True) + 1e-8)
    h = lax.dot_general(hist, w1_ref[...], (((1,), (1,)), ((), ())),
                        precision=lax.Precision.HIGHEST,
                        preferred_element_type=jnp.float32) + b1_ref[...]
    h = h * (1.0 / (1.0 + jnp.exp(-h)))
    o = lax.dot_general(h, w2_ref[...], (((1,), (1,)), ((), ())),
                        precision=lax.Precision.HIGHEST,
                        preferred_element_type=jnp.float32) + b2_ref[...]
    o_ref[...] = o


def kernel(pos, batch, W1, b1, W2, b2):
    pos_flat = jnp.concatenate(
        [pos, jnp.zeros((_L, 3), jnp.float32)]).reshape(3 * _NP)
    batch_p = jnp.concatenate([batch, jnp.zeros((_L,), jnp.int32)])
    starts = jnp.searchsorted(batch, jnp.arange(_NB + 1, dtype=jnp.int32))
    starts = jnp.concatenate(
        [starts.astype(jnp.int32), jnp.zeros((32 - (_NB + 1),), jnp.int32)])
    edges2 = jnp.array([(b * _MAX_DIST / _NUM_BINS) ** 2
                        for b in range(_NUM_BINS)], jnp.float32)
    parts = _sc_hist(pos_flat, batch_p, starts, edges2)
    parts2 = parts.reshape(_NW * _NB, _NUM_BINS)
    out = pl.pallas_call(
        _mlp_body,
        out_shape=jax.ShapeDtypeStruct((_NB, _HIDDEN), jnp.float32),
    )(parts2, W1, b1.reshape(1, _HIDDEN), W2, b2.reshape(1, _HIDDEN))
    return out
